# baseline (speedup 1.0000x reference)
import functools
import math

import jax
import jax.numpy as jnp
from jax.experimental import pallas as pl
from jax.experimental.pallas import tpu as pltpu

_C_LO = 128
_NUM_1O = 128
_KW = 3 * _NUM_1O
_TM = 1024
_VMEM_LIMIT = 48 * 1024 * 1024


def _readout_body(x_hbm, w_ref, o_ref, x_buf, in_sem, *, tm, n_steps):
    def dma_in(slot, step):
        pltpu.make_async_copy(
            x_hbm.at[pl.ds(step * tm, tm), pl.ds(_C_LO, _KW)],
            x_buf.at[slot], in_sem.at[slot]).start()

    def wait_in(slot):
        pltpu.make_async_copy(
            x_hbm.at[pl.ds(0, tm), pl.ds(_C_LO, _KW)],
            x_buf.at[slot], in_sem.at[slot]).wait()

    dma_in(0, 0)

    @pl.when(n_steps > 1)
    def _():
        dma_in(1, 1)

    @pl.when(n_steps > 2)
    def _():
        dma_in(2, 2)

    row_c = jax.lax.broadcasted_iota(jnp.int32, (_NUM_1O, _KW), 0)
    col_r = jax.lax.broadcasted_iota(jnp.int32, (_NUM_1O, _KW), 1)
    pattern = (col_r // 3 == row_c).astype(jnp.float32)
    wrep = jnp.dot(w_ref[...], pattern, preferred_element_type=jnp.float32,
                   precision=jax.lax.Precision.HIGHEST)

    r_i = jax.lax.broadcasted_iota(jnp.int32, (_KW, 8), 0)
    m_i = jax.lax.broadcasted_iota(jnp.int32, (_KW, 8), 1)
    mask = (r_i % 3 == m_i).astype(jnp.float32) * (1.0 / math.sqrt(float(_NUM_1O)))

    def body(step, _):
        cur = jax.lax.rem(step, 4)

        @pl.when(step + 3 < n_steps)
        def _():
            dma_in(jax.lax.rem(step + 3, 4), step + 3)

        wait_in(cur)
        y = x_buf[cur] * wrep
        acc = jnp.dot(y, mask, preferred_element_type=jnp.float32)
        o_ref[pl.ds(step * tm, tm), :] = acc[:, :3]
        return ()

    jax.lax.fori_loop(0, n_steps, body, (), unroll=False)


def kernel(x_flat, weight_1o):
    m, k = x_flat.shape
    num_1o = weight_1o.shape[0]
    assert k == _C_LO + 3 * _NUM_1O and num_1o == _NUM_1O

    tm = _TM
    while m % tm != 0:
        tm //= 2
    n_steps = m // tm

    wv = weight_1o.astype(jnp.float32).reshape(1, _NUM_1O)

    body = functools.partial(_readout_body, tm=tm, n_steps=n_steps)
    return pl.pallas_call(
        body,
        out_shape=jax.ShapeDtypeStruct((m, 3), jnp.float32),
        in_specs=[
            pl.BlockSpec(memory_space=pl.ANY),
            pl.BlockSpec((1, _NUM_1O), lambda: (0, 0)),
        ],
        out_specs=pl.BlockSpec((m, 3), lambda: (0, 0)),
        scratch_shapes=[
            pltpu.VMEM((4, tm, _KW), jnp.float32),
            pltpu.SemaphoreType.DMA((4,)),
        ],
        compiler_params=pltpu.CompilerParams(
            vmem_limit_bytes=_VMEM_LIMIT),
    )(x_flat, wv)


# speedup vs baseline: 1.0070x; 1.0070x over previous
import functools
import math

import jax
import jax.numpy as jnp
from jax.experimental import pallas as pl
from jax.experimental.pallas import tpu as pltpu

_C_LO = 128
_NUM_1O = 128
_KW = 3 * _NUM_1O
_TM = 2048
_VMEM_LIMIT = 48 * 1024 * 1024


def _readout_body(x_hbm, w_ref, o_ref, x_buf, in_sem, *, tm, n_steps):
    def dma_in(slot, step):
        pltpu.make_async_copy(
            x_hbm.at[pl.ds(step * tm, tm), pl.ds(_C_LO, _KW)],
            x_buf.at[slot], in_sem.at[slot]).start()

    def wait_in(slot):
        pltpu.make_async_copy(
            x_hbm.at[pl.ds(0, tm), pl.ds(_C_LO, _KW)],
            x_buf.at[slot], in_sem.at[slot]).wait()

    dma_in(0, 0)

    @pl.when(n_steps > 1)
    def _():
        dma_in(1, 1)

    @pl.when(n_steps > 2)
    def _():
        dma_in(2, 2)

    @pl.when(n_steps > 3)
    def _():
        dma_in(3, 3)

    @pl.when(n_steps > 4)
    def _():
        dma_in(4, 4)

    row_c = jax.lax.broadcasted_iota(jnp.int32, (_NUM_1O, _KW), 0)
    col_r = jax.lax.broadcasted_iota(jnp.int32, (_NUM_1O, _KW), 1)
    pattern = (col_r // 3 == row_c).astype(jnp.float32)
    wrep = jnp.dot(w_ref[...], pattern, preferred_element_type=jnp.float32,
                   precision=jax.lax.Precision.HIGHEST)

    r_i = jax.lax.broadcasted_iota(jnp.int32, (_KW, 8), 0)
    m_i = jax.lax.broadcasted_iota(jnp.int32, (_KW, 8), 1)
    mask = (r_i % 3 == m_i).astype(jnp.float32) * (1.0 / math.sqrt(float(_NUM_1O)))

    def body(step, _):
        cur = jax.lax.rem(step, 6)

        @pl.when(step + 5 < n_steps)
        def _():
            dma_in(jax.lax.rem(step + 5, 6), step + 5)

        wait_in(cur)
        y = x_buf[cur] * wrep
        acc = jnp.dot(y, mask, preferred_element_type=jnp.float32)
        o_ref[pl.ds(step * tm, tm), :] = acc[:, :3]
        return ()

    jax.lax.fori_loop(0, n_steps, body, (), unroll=False)


def kernel(x_flat, weight_1o):
    m, k = x_flat.shape
    num_1o = weight_1o.shape[0]
    assert k == _C_LO + 3 * _NUM_1O and num_1o == _NUM_1O

    tm = _TM
    while m % tm != 0:
        tm //= 2
    n_steps = m // tm

    wv = weight_1o.astype(jnp.float32).reshape(1, _NUM_1O)

    body = functools.partial(_readout_body, tm=tm, n_steps=n_steps)
    return pl.pallas_call(
        body,
        out_shape=jax.ShapeDtypeStruct((m, 3), jnp.float32),
        in_specs=[
            pl.BlockSpec(memory_space=pl.ANY),
            pl.BlockSpec((1, _NUM_1O), lambda: (0, 0)),
        ],
        out_specs=pl.BlockSpec((m, 3), lambda: (0, 0)),
        scratch_shapes=[
            pltpu.VMEM((6, tm, _KW), jnp.float32),
            pltpu.SemaphoreType.DMA((6,)),
        ],
        compiler_params=pltpu.CompilerParams(
            vmem_limit_bytes=_VMEM_LIMIT),
    )(x_flat, wv)


# speedup vs baseline: 1.0191x; 1.0119x over previous
import functools
import math

import jax
import jax.numpy as jnp
from jax.experimental import pallas as pl
from jax.experimental.pallas import tpu as pltpu

_C_LO = 128
_NUM_1O = 128
_KW = 3 * _NUM_1O
_TM = 2048
_VMEM_LIMIT = 48 * 1024 * 1024


def _readout_body(x_hbm, w_ref, o_ref, x_buf, in_sem, *, tm, n_steps):
    th = tm // 2

    def dma_in(slot, step):
        pltpu.make_async_copy(
            x_hbm.at[pl.ds(step * tm, th), pl.ds(_C_LO, _KW)],
            x_buf.at[slot, pl.ds(0, th)], in_sem.at[slot, 0]).start()
        pltpu.make_async_copy(
            x_hbm.at[pl.ds(step * tm + th, th), pl.ds(_C_LO, _KW)],
            x_buf.at[slot, pl.ds(th, th)], in_sem.at[slot, 1]).start()

    def wait_in(slot):
        pltpu.make_async_copy(
            x_hbm.at[pl.ds(0, th), pl.ds(_C_LO, _KW)],
            x_buf.at[slot, pl.ds(0, th)], in_sem.at[slot, 0]).wait()
        pltpu.make_async_copy(
            x_hbm.at[pl.ds(0, th), pl.ds(_C_LO, _KW)],
            x_buf.at[slot, pl.ds(th, th)], in_sem.at[slot, 1]).wait()

    dma_in(0, 0)

    @pl.when(n_steps > 1)
    def _():
        dma_in(1, 1)

    @pl.when(n_steps > 2)
    def _():
        dma_in(2, 2)

    row_c = jax.lax.broadcasted_iota(jnp.int32, (_NUM_1O, _KW), 0)
    col_r = jax.lax.broadcasted_iota(jnp.int32, (_NUM_1O, _KW), 1)
    pattern = (col_r // 3 == row_c).astype(jnp.float32)
    wrep = jnp.dot(w_ref[...], pattern, preferred_element_type=jnp.float32,
                   precision=jax.lax.Precision.HIGHEST)

    r_i = jax.lax.broadcasted_iota(jnp.int32, (_KW, 8), 0)
    m_i = jax.lax.broadcasted_iota(jnp.int32, (_KW, 8), 1)
    mask = (r_i % 3 == m_i).astype(jnp.float32) * (1.0 / math.sqrt(float(_NUM_1O)))

    def body(step, _):
        cur = jax.lax.rem(step, 4)

        @pl.when(step + 3 < n_steps)
        def _():
            dma_in(jax.lax.rem(step + 3, 4), step + 3)

        wait_in(cur)
        y = x_buf[cur] * wrep
        acc = jnp.dot(y, mask, preferred_element_type=jnp.float32)
        o_ref[pl.ds(step * tm, tm), :] = acc[:, :3]
        return ()

    jax.lax.fori_loop(0, n_steps, body, (), unroll=False)


def kernel(x_flat, weight_1o):
    m, k = x_flat.shape
    num_1o = weight_1o.shape[0]
    assert k == _C_LO + 3 * _NUM_1O and num_1o == _NUM_1O

    tm = _TM
    while m % tm != 0:
        tm //= 2
    n_steps = m // tm

    wv = weight_1o.astype(jnp.float32).reshape(1, _NUM_1O)

    body = functools.partial(_readout_body, tm=tm, n_steps=n_steps)
    return pl.pallas_call(
        body,
        out_shape=jax.ShapeDtypeStruct((m, 3), jnp.float32),
        in_specs=[
            pl.BlockSpec(memory_space=pl.ANY),
            pl.BlockSpec((1, _NUM_1O), lambda: (0, 0)),
        ],
        out_specs=pl.BlockSpec((m, 3), lambda: (0, 0)),
        scratch_shapes=[
            pltpu.VMEM((4, tm, _KW), jnp.float32),
            pltpu.SemaphoreType.DMA((4, 2)),
        ],
        compiler_params=pltpu.CompilerParams(
            vmem_limit_bytes=_VMEM_LIMIT),
    )(x_flat, wv)
